# baseline (speedup 1.0000x reference)
import jax
import jax.numpy as jnp
from jax import lax
from jax.experimental import pallas as pl
from jax.experimental.pallas import tpu as pltpu
from jax.experimental.pallas import tpu_sc as plsc

B = 16384
D = 64
N = 100000

NC = 2
NS = 16
SAMP = B // NS

C = 16896
NCHUNKS_PER_CORE = 3
TRASH = 256
ROWS_PER_TILE = C // NS
FBLK = 32
ZROWS = 64
MAX_INFLIGHT = 8

T_OFF = 0
CNT_OFF = 16
ONES_OFF = 48
Z1D_OFF = 176
MISC_LEN = 2048


def _sc_body(x_hbm, y_hbm, t_hbm, ctr_hbm, out_hbm,
             x_v, y_v, idx2d, z2, misc,
             sum_buf, ctr_buf, out_buf,
             sums_sh, cnts_sh, zsem):
    c = lax.axis_index("c")
    s = lax.axis_index("s")

    pltpu.sync_copy(y_hbm.at[pl.ds(s * SAMP, SAMP)], y_v)
    pltpu.sync_copy(t_hbm, misc.at[pl.ds(T_OFF, 16)])

    zeros16 = jnp.zeros((16,), jnp.float32)
    for r in range(ZROWS):
        for q in range(4):
            z2[r, pl.ds(q * 16, 16)] = zeros16
    for i in range(8):
        misc[pl.ds(ONES_OFF + i * 16, 16)] = zeros16 + 1.0

    def _fill_z1d(i, carry):
        misc[pl.ds(Z1D_OFF + i * 16, 16)] = zeros16
        return carry
    lax.fori_loop(0, ROWS_PER_TILE // 16, _fill_z1d, 0)

    tv = misc[pl.ds(T_OFF, 16)]
    itv = 1.0 / (tv + 1.0)
    lane = lax.broadcasted_iota(jnp.int32, (16,), 0)

    for j in range(NCHUNKS_PER_CORE):
        base = (NCHUNKS_PER_CORE * c + j) * C
        n_rows = jnp.minimum(jnp.int32(C), jnp.int32(N) - base)

        zdescs = []
        rbase = s * ROWS_PER_TILE
        for k in range(ROWS_PER_TILE // ZROWS):
            zdescs.append(pltpu.async_copy(
                z2, sums_sh.at[pl.ds(rbase + k * ZROWS, ZROWS)], zsem))
            if len(zdescs) >= MAX_INFLIGHT:
                for dsc in zdescs:
                    dsc.wait()
                zdescs = []
        tail = ROWS_PER_TILE - (ROWS_PER_TILE // ZROWS) * ZROWS
        if tail:
            zdescs.append(pltpu.async_copy(
                z2.at[pl.ds(0, tail)],
                sums_sh.at[pl.ds(rbase + ROWS_PER_TILE - tail, tail)], zsem))
        zdescs.append(pltpu.async_copy(
            misc.at[pl.ds(Z1D_OFF, ROWS_PER_TILE)],
            cnts_sh.at[pl.ds(rbase, ROWS_PER_TILE)], zsem))
        for dsc in zdescs:
            dsc.wait()
        plsc.subcore_barrier()

        for i in range(SAMP // 16):
            yv = y_v[pl.ds(i * 16, 16)]
            rel = yv - base
            inb = (rel >= 0) & (rel < n_rows)
            trash = C + (((s + i) * 16 + lane) & (TRASH - 1))
            idx2d[i // 8, pl.ds((i % 8) * 16, 16)] = jnp.where(inb, rel, trash)
        for jd in range(SAMP // 128):
            pltpu.sync_copy(x_hbm.at[pl.ds(s * SAMP + jd * 128, 128)], x_v)
            pltpu.sync_copy(x_v, sums_sh.at[idx2d.at[jd]], add=True)
            pltpu.sync_copy(misc.at[pl.ds(ONES_OFF, 128)],
                            cnts_sh.at[idx2d.at[jd]], add=True)
        plsc.subcore_barrier()

        n_blocks = n_rows // FBLK
        num_my = (n_blocks + 15 - s) // 16

        def _blk(k, carry):
            r = (s + 16 * k) * FBLK
            g = base + r
            pltpu.sync_copy(cnts_sh.at[pl.ds(r, FBLK)],
                            misc.at[pl.ds(CNT_OFF, FBLK)])
            pltpu.sync_copy(sums_sh.at[pl.ds(r, FBLK)], sum_buf)
            pltpu.sync_copy(ctr_hbm.at[pl.ds(g, FBLK)], ctr_buf)
            for h in range(FBLK // 16):
                cv = misc[pl.ds(CNT_OFF + h * 16, 16)]
                pres = cv > 0.0
                a_vec = jnp.where(pres, itv / jnp.maximum(cv, 1.0), 0.0)
                b_vec = jnp.where(pres, tv * itv, 1.0)
                for rr in range(16):
                    row = h * 16 + rr
                    av = jnp.full((16,), a_vec[rr])
                    bv = jnp.full((16,), b_vec[rr])
                    for q in range(D // 16):
                        sl = pl.ds(q * 16, 16)
                        out_buf[row, sl] = (av * sum_buf[row, sl]
                                            + bv * ctr_buf[row, sl])
            pltpu.sync_copy(out_buf, out_hbm.at[pl.ds(g, FBLK)])
            return carry
        lax.fori_loop(0, num_my, _blk, 0)
        plsc.subcore_barrier()


@jax.jit
def _run(x, y, t16, centers):
    mesh = plsc.VectorSubcoreMesh(core_axis_name="c", subcore_axis_name="s")
    f = pl.kernel(
        _sc_body,
        out_type=jax.ShapeDtypeStruct((N, D), jnp.float32),
        mesh=mesh,
        scratch_types=[
            pltpu.VMEM((128, D), jnp.float32),
            pltpu.VMEM((SAMP,), jnp.int32),
            pltpu.VMEM((SAMP // 128, 128), jnp.int32),
            pltpu.VMEM((ZROWS, D), jnp.float32),
            pltpu.VMEM((MISC_LEN,), jnp.float32),
            pltpu.VMEM((FBLK, D), jnp.float32),
            pltpu.VMEM((FBLK, D), jnp.float32),
            pltpu.VMEM((FBLK, D), jnp.float32),
            pltpu.VMEM_SHARED((C + TRASH, D), jnp.float32),
            pltpu.VMEM_SHARED((C + TRASH,), jnp.float32),
            pltpu.SemaphoreType.DMA,
        ],
        compiler_params=pltpu.CompilerParams(use_tc_tiling_on_sc=False),
    )
    return f(x, y, t16, centers)


def kernel(x, y, centers, counter):
    assert x.shape == (B, D) and centers.shape == (N, D)
    t16 = jnp.broadcast_to(counter.astype(jnp.float32), (16,))
    new_centers = _run(x, y.astype(jnp.int32), t16, centers)
    return new_centers, counter + 1.0


# speedup vs baseline: 1.0683x; 1.0683x over previous
import jax
import jax.numpy as jnp
from jax import lax
from jax.experimental import pallas as pl
from jax.experimental.pallas import tpu as pltpu
from jax.experimental.pallas import tpu_sc as plsc

B = 16384
D = 64
N = 100000

NC = 2
NS = 16
SAMP = B // NS

C = 16896
NCHUNKS_PER_CORE = 3
TRASH = 256
ROWS_PER_TILE = C // NS
FBLK = 32
ZROWS = 64
MAX_INFLIGHT = 8

T_OFF = 0
ONES_OFF = 16
Z1D_OFF = 144
CNT_OFF = 1200
MISC_LEN = 4096


def _sc_body(x_hbm, y_hbm, t_hbm, ctr_hbm, out_hbm,
             x_v0, x_v1, y_v, idx2d, z2, misc,
             sum0, sum1, ctr0, ctr1, outb0, outb1,
             sums_sh, cnts_sh,
             zsem, xsem, fsem0, fsem1, osem0, osem1):
    sum_b = (sum0, sum1)
    ctr_b = (ctr0, ctr1)
    out_b = (outb0, outb1)
    c = lax.axis_index("c")
    s = lax.axis_index("s")
    x_bufs = (x_v0, x_v1)
    fsems = (fsem0, fsem1)
    osems = (osem0, osem1)

    pltpu.sync_copy(y_hbm.at[pl.ds(s * SAMP, SAMP)], y_v)
    pltpu.sync_copy(t_hbm, misc.at[pl.ds(T_OFF, 16)])

    zeros16 = jnp.zeros((16,), jnp.float32)
    for r in range(ZROWS):
        for q in range(4):
            z2[r, pl.ds(q * 16, 16)] = zeros16
    for i in range(8):
        misc[pl.ds(ONES_OFF + i * 16, 16)] = zeros16 + 1.0

    def _fill_z1d(i, carry):
        misc[pl.ds(Z1D_OFF + i * 16, 16)] = zeros16
        return carry
    lax.fori_loop(0, ROWS_PER_TILE // 16, _fill_z1d, 0)

    tv = misc[pl.ds(T_OFF, 16)]
    itv = 1.0 / (tv + 1.0)
    lane = lax.broadcasted_iota(jnp.int32, (16,), 0)

    for j in range(NCHUNKS_PER_CORE):
        base = (NCHUNKS_PER_CORE * c + j) * C
        n_rows = jnp.minimum(jnp.int32(C), jnp.int32(N) - base)

        zdescs = []
        rbase = s * ROWS_PER_TILE
        for k in range(ROWS_PER_TILE // ZROWS):
            zdescs.append(pltpu.async_copy(
                z2, sums_sh.at[pl.ds(rbase + k * ZROWS, ZROWS)], zsem))
            if len(zdescs) >= MAX_INFLIGHT:
                for dsc in zdescs:
                    dsc.wait()
                zdescs = []
        tail = ROWS_PER_TILE - (ROWS_PER_TILE // ZROWS) * ZROWS
        if tail:
            zdescs.append(pltpu.async_copy(
                z2.at[pl.ds(0, tail)],
                sums_sh.at[pl.ds(rbase + ROWS_PER_TILE - tail, tail)], zsem))
        zdescs.append(pltpu.async_copy(
            misc.at[pl.ds(Z1D_OFF, ROWS_PER_TILE)],
            cnts_sh.at[pl.ds(rbase, ROWS_PER_TILE)], zsem))
        for dsc in zdescs:
            dsc.wait()
        plsc.subcore_barrier()

        for i in range(SAMP // 16):
            yv = y_v[pl.ds(i * 16, 16)]
            rel = yv - base
            inb = (rel >= 0) & (rel < n_rows)
            trash = C + (((s + i) * 16 + lane) & (TRASH - 1))
            idx2d[i // 8, pl.ds((i % 8) * 16, 16)] = jnp.where(inb, rel, trash)
        nxb = SAMP // 128
        xd = [None] * nxb
        xd[0] = pltpu.async_copy(x_hbm.at[pl.ds(s * SAMP, 128)], x_v0, xsem)
        for jd in range(nxb):
            xd[jd].wait()
            if jd + 1 < nxb:
                xd[jd + 1] = pltpu.async_copy(
                    x_hbm.at[pl.ds(s * SAMP + (jd + 1) * 128, 128)],
                    x_bufs[(jd + 1) % 2], xsem)
            pltpu.sync_copy(x_bufs[jd % 2], sums_sh.at[idx2d.at[jd]], add=True)
            pltpu.sync_copy(misc.at[pl.ds(ONES_OFF, 128)],
                            cnts_sh.at[idx2d.at[jd]], add=True)
        plsc.subcore_barrier()

        n_blocks = n_rows // FBLK
        b0 = (s * n_blocks) // NS
        b1 = ((s + 1) * n_blocks) // NS
        nmy = b1 - b0

        pltpu.sync_copy(cnts_sh.at[pl.ds(b0 * FBLK, ROWS_PER_TILE)],
                        misc.at[pl.ds(CNT_OFF, ROWS_PER_TILE)])

        def compute(slot, k):
            kk = k - b0
            for h in range(FBLK // 16):
                cv = misc[pl.ds(CNT_OFF + kk * FBLK + h * 16, 16)]
                pres = cv > 0.0
                a_vec = jnp.where(pres, itv / jnp.maximum(cv, 1.0), 0.0)
                b_vec = jnp.where(pres, tv * itv, 1.0)
                for rr in range(16):
                    row = h * 16 + rr
                    av = jnp.full((16,), a_vec[rr])
                    bv = jnp.full((16,), b_vec[rr])
                    for q in range(D // 16):
                        sl = pl.ds(q * 16, 16)
                        out_b[slot][row, sl] = (av * sum_b[slot][row, sl]
                                                + bv * ctr_b[slot][row, sl])

        def _blk(i, carry):
            k = b0 + i
            pltpu.sync_copy(sums_sh.at[pl.ds(k * FBLK, FBLK)], sum_b[0])
            pltpu.sync_copy(ctr_hbm.at[pl.ds(base + k * FBLK, FBLK)],
                            ctr_b[0])
            compute(0, k)
            pltpu.sync_copy(out_b[0],
                            out_hbm.at[pl.ds(base + k * FBLK, FBLK)])
            return carry
        lax.fori_loop(0, nmy, _blk, 0)
        plsc.subcore_barrier()


@jax.jit
def _run(x, y, t16, centers):
    mesh = plsc.VectorSubcoreMesh(core_axis_name="c", subcore_axis_name="s")
    f = pl.kernel(
        _sc_body,
        out_type=jax.ShapeDtypeStruct((N, D), jnp.float32),
        mesh=mesh,
        scratch_types=[
            pltpu.VMEM((128, D), jnp.float32),
            pltpu.VMEM((128, D), jnp.float32),
            pltpu.VMEM((SAMP,), jnp.int32),
            pltpu.VMEM((SAMP // 128, 128), jnp.int32),
            pltpu.VMEM((ZROWS, D), jnp.float32),
            pltpu.VMEM((MISC_LEN,), jnp.float32),
            pltpu.VMEM((FBLK, D), jnp.float32),
            pltpu.VMEM((FBLK, D), jnp.float32),
            pltpu.VMEM((FBLK, D), jnp.float32),
            pltpu.VMEM((FBLK, D), jnp.float32),
            pltpu.VMEM((FBLK, D), jnp.float32),
            pltpu.VMEM((FBLK, D), jnp.float32),
            pltpu.VMEM_SHARED((C + TRASH, D), jnp.float32),
            pltpu.VMEM_SHARED((C + TRASH,), jnp.float32),
            pltpu.SemaphoreType.DMA,
            pltpu.SemaphoreType.DMA,
            pltpu.SemaphoreType.DMA,
            pltpu.SemaphoreType.DMA,
            pltpu.SemaphoreType.DMA,
            pltpu.SemaphoreType.DMA,
        ],
        compiler_params=pltpu.CompilerParams(use_tc_tiling_on_sc=False),
    )
    return f(x, y, t16, centers)


def kernel(x, y, centers, counter):
    assert x.shape == (B, D) and centers.shape == (N, D)
    t16 = jnp.broadcast_to(counter.astype(jnp.float32), (16,))
    new_centers = _run(x, y.astype(jnp.int32), t16, centers)
    return new_centers, counter + 1.0


# speedup vs baseline: 1.2561x; 1.1757x over previous
import jax
import jax.numpy as jnp
from jax import lax
from jax.experimental import pallas as pl
from jax.experimental.pallas import tpu as pltpu
from jax.experimental.pallas import tpu_sc as plsc

B = 16384
D = 64
N = 100000

NC = 2
NS = 16
SAMP = B // NS

C = 16896
NCHUNKS_PER_CORE = 3
TRASH = 256
ROWS_PER_TILE = C // NS
FBLK = 96
REM = 32
ZROWS = 64
MAX_INFLIGHT = 8

T_OFF = 0
ONES_OFF = 16
Z1D_OFF = 144
CNT_OFF = 1200
MISC_LEN = 4096


def _sc_body(x_hbm, y_hbm, t_hbm, ctr_hbm, out_hbm,
             x_v0, x_v1, y_v, idx2d, z2, misc,
             sum0, sum1, ctr0, ctr1, outb0, outb1,
             sums_sh, cnts_sh,
             zsem, xsem, fsem0, fsem1, osem0, osem1):
    sum_b = (sum0, sum1)
    ctr_b = (ctr0, ctr1)
    out_b = (outb0, outb1)
    c = lax.axis_index("c")
    s = lax.axis_index("s")
    x_bufs = (x_v0, x_v1)
    fsems = (fsem0, fsem1)
    osems = (osem0, osem1)

    pltpu.sync_copy(y_hbm.at[pl.ds(s * SAMP, SAMP)], y_v)
    pltpu.sync_copy(t_hbm, misc.at[pl.ds(T_OFF, 16)])

    zeros16 = jnp.zeros((16,), jnp.float32)
    for r in range(ZROWS):
        for q in range(4):
            z2[r, pl.ds(q * 16, 16)] = zeros16
    for i in range(8):
        misc[pl.ds(ONES_OFF + i * 16, 16)] = zeros16 + 1.0

    def _fill_z1d(i, carry):
        misc[pl.ds(Z1D_OFF + i * 16, 16)] = zeros16
        return carry
    lax.fori_loop(0, ROWS_PER_TILE // 16, _fill_z1d, 0)

    tv = misc[pl.ds(T_OFF, 16)]
    itv = 1.0 / (tv + 1.0)
    lane = lax.broadcasted_iota(jnp.int32, (16,), 0)

    for j in range(NCHUNKS_PER_CORE):
        base = (NCHUNKS_PER_CORE * c + j) * C
        n_rows = jnp.minimum(jnp.int32(C), jnp.int32(N) - base)

        zdescs = []
        rbase = s * ROWS_PER_TILE
        for k in range(ROWS_PER_TILE // ZROWS):
            zdescs.append(pltpu.async_copy(
                z2, sums_sh.at[pl.ds(rbase + k * ZROWS, ZROWS)], zsem))
            if len(zdescs) >= MAX_INFLIGHT:
                for dsc in zdescs:
                    dsc.wait()
                zdescs = []
        tail = ROWS_PER_TILE - (ROWS_PER_TILE // ZROWS) * ZROWS
        if tail:
            zdescs.append(pltpu.async_copy(
                z2.at[pl.ds(0, tail)],
                sums_sh.at[pl.ds(rbase + ROWS_PER_TILE - tail, tail)], zsem))
        zdescs.append(pltpu.async_copy(
            misc.at[pl.ds(Z1D_OFF, ROWS_PER_TILE)],
            cnts_sh.at[pl.ds(rbase, ROWS_PER_TILE)], zsem))
        for dsc in zdescs:
            dsc.wait()
        plsc.subcore_barrier()

        for i in range(SAMP // 16):
            yv = y_v[pl.ds(i * 16, 16)]
            rel = yv - base
            inb = (rel >= 0) & (rel < n_rows)
            trash = C + (((s + i) * 16 + lane) & (TRASH - 1))
            idx2d[i // 8, pl.ds((i % 8) * 16, 16)] = jnp.where(inb, rel, trash)
        nxb = SAMP // 128
        xd = [None] * nxb
        xd[0] = pltpu.async_copy(x_hbm.at[pl.ds(s * SAMP, 128)], x_v0, xsem)
        for jd in range(nxb):
            xd[jd].wait()
            if jd + 1 < nxb:
                xd[jd + 1] = pltpu.async_copy(
                    x_hbm.at[pl.ds(s * SAMP + (jd + 1) * 128, 128)],
                    x_bufs[(jd + 1) % 2], xsem)
            pltpu.sync_copy(x_bufs[jd % 2], sums_sh.at[idx2d.at[jd]], add=True)
            pltpu.sync_copy(misc.at[pl.ds(ONES_OFF, 128)],
                            cnts_sh.at[idx2d.at[jd]], add=True)
        plsc.subcore_barrier()

        n_blocks = n_rows // FBLK
        b0 = (s * n_blocks) // NS
        b1 = ((s + 1) * n_blocks) // NS
        nmy = b1 - b0

        pltpu.sync_copy(cnts_sh.at[pl.ds(b0 * FBLK, ROWS_PER_TILE)],
                        misc.at[pl.ds(CNT_OFF, ROWS_PER_TILE)])

        def compute(rows, coff):
            for h in range(rows // 16):
                cv = misc[pl.ds(CNT_OFF + coff + h * 16, 16)]
                pres = cv > 0.0
                a_vec = jnp.where(pres, itv / jnp.maximum(cv, 1.0), 0.0)
                b_vec = jnp.where(pres, tv * itv, 1.0)
                for rr in range(16):
                    row = h * 16 + rr
                    av = jnp.full((16,), a_vec[rr])
                    bv = jnp.full((16,), b_vec[rr])
                    for q in range(D // 16):
                        sl = pl.ds(q * 16, 16)
                        out_b[0][row, sl] = (av * sum_b[0][row, sl]
                                             + bv * ctr_b[0][row, sl])

        def _blk(i, carry):
            k = b0 + i
            pltpu.sync_copy(sums_sh.at[pl.ds(k * FBLK, FBLK)], sum_b[0])
            pltpu.sync_copy(ctr_hbm.at[pl.ds(base + k * FBLK, FBLK)],
                            ctr_b[0])
            compute(FBLK, (k - b0) * FBLK)
            pltpu.sync_copy(out_b[0],
                            out_hbm.at[pl.ds(base + k * FBLK, FBLK)])
            return carry
        lax.fori_loop(0, nmy, _blk, 0)

        rem32 = (n_rows - n_blocks * FBLK) // REM

        def _remblk(i, carry):
            r = n_blocks * FBLK + s * REM
            pltpu.sync_copy(cnts_sh.at[pl.ds(r, REM)],
                            misc.at[pl.ds(CNT_OFF, REM)])
            pltpu.sync_copy(sums_sh.at[pl.ds(r, REM)],
                            sum_b[0].at[pl.ds(0, REM)])
            pltpu.sync_copy(ctr_hbm.at[pl.ds(base + r, REM)],
                            ctr_b[0].at[pl.ds(0, REM)])
            compute(REM, 0)
            pltpu.sync_copy(out_b[0].at[pl.ds(0, REM)],
                            out_hbm.at[pl.ds(base + r, REM)])
            return carry
        lax.fori_loop(0, jnp.where(s < rem32, 1, 0), _remblk, 0)
        plsc.subcore_barrier()


@jax.jit
def _run(x, y, t16, centers):
    mesh = plsc.VectorSubcoreMesh(core_axis_name="c", subcore_axis_name="s")
    f = pl.kernel(
        _sc_body,
        out_type=jax.ShapeDtypeStruct((N, D), jnp.float32),
        mesh=mesh,
        scratch_types=[
            pltpu.VMEM((128, D), jnp.float32),
            pltpu.VMEM((128, D), jnp.float32),
            pltpu.VMEM((SAMP,), jnp.int32),
            pltpu.VMEM((SAMP // 128, 128), jnp.int32),
            pltpu.VMEM((ZROWS, D), jnp.float32),
            pltpu.VMEM((MISC_LEN,), jnp.float32),
            pltpu.VMEM((FBLK, D), jnp.float32),
            pltpu.VMEM((FBLK, D), jnp.float32),
            pltpu.VMEM((FBLK, D), jnp.float32),
            pltpu.VMEM((FBLK, D), jnp.float32),
            pltpu.VMEM((FBLK, D), jnp.float32),
            pltpu.VMEM((FBLK, D), jnp.float32),
            pltpu.VMEM_SHARED((C + TRASH, D), jnp.float32),
            pltpu.VMEM_SHARED((C + TRASH,), jnp.float32),
            pltpu.SemaphoreType.DMA,
            pltpu.SemaphoreType.DMA,
            pltpu.SemaphoreType.DMA,
            pltpu.SemaphoreType.DMA,
            pltpu.SemaphoreType.DMA,
            pltpu.SemaphoreType.DMA,
        ],
        compiler_params=pltpu.CompilerParams(use_tc_tiling_on_sc=False),
    )
    return f(x, y, t16, centers)


def kernel(x, y, centers, counter):
    assert x.shape == (B, D) and centers.shape == (N, D)
    t16 = jnp.broadcast_to(counter.astype(jnp.float32), (16,))
    new_centers = _run(x, y.astype(jnp.int32), t16, centers)
    return new_centers, counter + 1.0


# speedup vs baseline: 1.2568x; 1.0006x over previous
import jax
import jax.numpy as jnp
from jax import lax
from jax.experimental import pallas as pl
from jax.experimental.pallas import tpu as pltpu
from jax.experimental.pallas import tpu_sc as plsc

B = 16384
D = 64
N = 100000

NC = 2
NS = 16
SAMP = B // NS

C = 16896
NCHUNKS_PER_CORE = 3
TRASH = 256
ROWS_PER_TILE = C // NS
FBLK = 128
REM = 32
CNT_SPAN = 1152
ZROWS = 64
MAX_INFLIGHT = 8

T_OFF = 0
ONES_OFF = 16
Z1D_OFF = 144
CNT_OFF = 1200
MISC_LEN = 4096


def _sc_body(x_hbm, y_hbm, t_hbm, ctr_hbm, out_hbm,
             x_v0, x_v1, y_v, idx2d, z2, misc,
             sum0, ctr0, outb0,
             sums_sh, cnts_sh,
             zsem, xsem, ssem, csem):
    sum_b = (sum0,)
    ctr_b = (ctr0,)
    out_b = (outb0,)
    c = lax.axis_index("c")
    s = lax.axis_index("s")
    x_bufs = (x_v0, x_v1)

    pltpu.sync_copy(y_hbm.at[pl.ds(s * SAMP, SAMP)], y_v)
    pltpu.sync_copy(t_hbm, misc.at[pl.ds(T_OFF, 16)])

    zeros16 = jnp.zeros((16,), jnp.float32)
    for r in range(ZROWS):
        for q in range(4):
            z2[r, pl.ds(q * 16, 16)] = zeros16
    for i in range(8):
        misc[pl.ds(ONES_OFF + i * 16, 16)] = zeros16 + 1.0

    def _fill_z1d(i, carry):
        misc[pl.ds(Z1D_OFF + i * 16, 16)] = zeros16
        return carry
    lax.fori_loop(0, ROWS_PER_TILE // 16, _fill_z1d, 0)

    tv = misc[pl.ds(T_OFF, 16)]
    itv = 1.0 / (tv + 1.0)
    lane = lax.broadcasted_iota(jnp.int32, (16,), 0)

    for j in range(NCHUNKS_PER_CORE):
        base = (NCHUNKS_PER_CORE * c + j) * C
        n_rows = jnp.minimum(jnp.int32(C), jnp.int32(N) - base)

        zdescs = []
        rbase = s * ROWS_PER_TILE
        for k in range(ROWS_PER_TILE // ZROWS):
            zdescs.append(pltpu.async_copy(
                z2, sums_sh.at[pl.ds(rbase + k * ZROWS, ZROWS)], zsem))
            if len(zdescs) >= MAX_INFLIGHT:
                for dsc in zdescs:
                    dsc.wait()
                zdescs = []
        tail = ROWS_PER_TILE - (ROWS_PER_TILE // ZROWS) * ZROWS
        if tail:
            zdescs.append(pltpu.async_copy(
                z2.at[pl.ds(0, tail)],
                sums_sh.at[pl.ds(rbase + ROWS_PER_TILE - tail, tail)], zsem))
        zdescs.append(pltpu.async_copy(
            misc.at[pl.ds(Z1D_OFF, ROWS_PER_TILE)],
            cnts_sh.at[pl.ds(rbase, ROWS_PER_TILE)], zsem))
        for dsc in zdescs:
            dsc.wait()
        plsc.subcore_barrier()

        for i in range(SAMP // 16):
            yv = y_v[pl.ds(i * 16, 16)]
            rel = yv - base
            inb = (rel >= 0) & (rel < n_rows)
            trash = C + (((s + i) * 16 + lane) & (TRASH - 1))
            idx2d[i // 8, pl.ds((i % 8) * 16, 16)] = jnp.where(inb, rel, trash)
        nxb = SAMP // 128
        xd = [None] * nxb
        sd = [None] * nxb
        cd = [None] * nxb
        xd[0] = pltpu.async_copy(x_hbm.at[pl.ds(s * SAMP, 128)], x_v0, xsem)
        for jd in range(nxb):
            xd[jd].wait()
            if jd + 1 < nxb:
                if jd >= 1:
                    sd[jd - 1].wait()
                xd[jd + 1] = pltpu.async_copy(
                    x_hbm.at[pl.ds(s * SAMP + (jd + 1) * 128, 128)],
                    x_bufs[(jd + 1) % 2], xsem)
            sd[jd] = pltpu.async_copy(x_bufs[jd % 2],
                                      sums_sh.at[idx2d.at[jd]], ssem,
                                      add=True)
            cd[jd] = pltpu.async_copy(misc.at[pl.ds(ONES_OFF, 128)],
                                      cnts_sh.at[idx2d.at[jd]], csem,
                                      add=True)
        sd[nxb - 2].wait()
        sd[nxb - 1].wait()
        for jd in range(nxb):
            cd[jd].wait()
        plsc.subcore_barrier()

        n_blocks = n_rows // FBLK
        b0 = (s * n_blocks) // NS
        b1 = ((s + 1) * n_blocks) // NS
        nmy = b1 - b0

        pltpu.sync_copy(cnts_sh.at[pl.ds(b0 * FBLK, CNT_SPAN)],
                        misc.at[pl.ds(CNT_OFF, CNT_SPAN)])

        def compute(rows, coff):
            for h in range(rows // 16):
                cv = misc[pl.ds(CNT_OFF + coff + h * 16, 16)]
                pres = cv > 0.0
                a_vec = jnp.where(pres, itv / jnp.maximum(cv, 1.0), 0.0)
                b_vec = jnp.where(pres, tv * itv, 1.0)
                for rr in range(16):
                    row = h * 16 + rr
                    av = jnp.full((16,), a_vec[rr])
                    bv = jnp.full((16,), b_vec[rr])
                    for q in range(D // 16):
                        sl = pl.ds(q * 16, 16)
                        out_b[0][row, sl] = (av * sum_b[0][row, sl]
                                             + bv * ctr_b[0][row, sl])

        def _blk(i, carry):
            k = b0 + i
            pltpu.sync_copy(sums_sh.at[pl.ds(k * FBLK, FBLK)], sum_b[0])
            pltpu.sync_copy(ctr_hbm.at[pl.ds(base + k * FBLK, FBLK)],
                            ctr_b[0])
            compute(FBLK, (k - b0) * FBLK)
            pltpu.sync_copy(out_b[0],
                            out_hbm.at[pl.ds(base + k * FBLK, FBLK)])
            return carry
        lax.fori_loop(0, nmy, _blk, 0)

        rem32 = (n_rows - n_blocks * FBLK) // REM

        def _remblk(i, carry):
            r = n_blocks * FBLK + s * REM
            pltpu.sync_copy(cnts_sh.at[pl.ds(r, REM)],
                            misc.at[pl.ds(CNT_OFF, REM)])
            pltpu.sync_copy(sums_sh.at[pl.ds(r, REM)],
                            sum_b[0].at[pl.ds(0, REM)])
            pltpu.sync_copy(ctr_hbm.at[pl.ds(base + r, REM)],
                            ctr_b[0].at[pl.ds(0, REM)])
            compute(REM, 0)
            pltpu.sync_copy(out_b[0].at[pl.ds(0, REM)],
                            out_hbm.at[pl.ds(base + r, REM)])
            return carry
        lax.fori_loop(0, jnp.where(s < rem32, 1, 0), _remblk, 0)
        plsc.subcore_barrier()


@jax.jit
def _run(x, y, t16, centers):
    mesh = plsc.VectorSubcoreMesh(core_axis_name="c", subcore_axis_name="s")
    f = pl.kernel(
        _sc_body,
        out_type=jax.ShapeDtypeStruct((N, D), jnp.float32),
        mesh=mesh,
        scratch_types=[
            pltpu.VMEM((128, D), jnp.float32),
            pltpu.VMEM((128, D), jnp.float32),
            pltpu.VMEM((SAMP,), jnp.int32),
            pltpu.VMEM((SAMP // 128, 128), jnp.int32),
            pltpu.VMEM((ZROWS, D), jnp.float32),
            pltpu.VMEM((MISC_LEN,), jnp.float32),
            pltpu.VMEM((FBLK, D), jnp.float32),
            pltpu.VMEM((FBLK, D), jnp.float32),
            pltpu.VMEM((FBLK, D), jnp.float32),
            pltpu.VMEM_SHARED((C + TRASH, D), jnp.float32),
            pltpu.VMEM_SHARED((C + TRASH,), jnp.float32),
            pltpu.SemaphoreType.DMA,
            pltpu.SemaphoreType.DMA,
            pltpu.SemaphoreType.DMA,
            pltpu.SemaphoreType.DMA,
        ],
        compiler_params=pltpu.CompilerParams(use_tc_tiling_on_sc=False),
    )
    return f(x, y, t16, centers)


def kernel(x, y, centers, counter):
    assert x.shape == (B, D) and centers.shape == (N, D)
    t16 = jnp.broadcast_to(counter.astype(jnp.float32), (16,))
    new_centers = _run(x, y.astype(jnp.int32), t16, centers)
    return new_centers, counter + 1.0
